# Initial kernel scaffold; baseline (speedup 1.0000x reference)
import functools

import jax
import jax.numpy as jnp
from jax import lax
from jax.experimental import pallas as pl
from jax.experimental.pallas import tpu as pltpu
from jax.experimental.pallas import tpu_sc as plsc

M = 30000
NV = 20000
NN = M + NV
DIM = 64
B = 4096
NEG = 40
REG = 1e-4

NW = 32
BPW = B // NW
CHUNK = 128
NCH_N = NEG
NCH = NCH_N + 4
NB = 4


def _sc_gather_factory():
    mesh = plsc.VectorSubcoreMesh(core_axis_name="c", subcore_axis_name="s")

    @functools.partial(
        pl.kernel,
        out_type=jax.ShapeDtypeStruct((NW, NCH, CHUNK, DIM), jnp.float32),
        mesh=mesh,
        scratch_types=[
            pltpu.VMEM((NCH, CHUNK), jnp.int32),
            pltpu.VMEM((NB, CHUNK, DIM), jnp.float32),
            pltpu.SemaphoreType.DMA((NB,)),
        ],
    )
    def sc_gather(idx_hbm, z_hbm, out_hbm, idx_v, rows_v, gsem):
        wid = lax.axis_index("s") * 2 + lax.axis_index("c")
        pltpu.sync_copy(idx_hbm.at[wid], idx_v)

        def start(g, b):
            pltpu.async_copy(z_hbm.at[idx_v.at[g]], rows_v.at[b], gsem.at[b])

        def wait(b):
            pltpu.make_async_copy(
                z_hbm.at[idx_v.at[0]], rows_v.at[b], gsem.at[b]
            ).wait()

        for b in range(NB):
            start(b, b)

        def group(i, carry):
            for b in range(NB):
                g = i * NB + b
                wait(b)
                pltpu.sync_copy(rows_v.at[b], out_hbm.at[wid, g])
                start(g + NB, b)
            return carry

        lax.fori_loop(0, NCH_N // NB, group, 0, unroll=False)

        for b in range(NB):
            g = NCH_N + b
            wait(b)
            if g < NCH_N + 2:
                pltpu.sync_copy(rows_v.at[b], out_hbm.at[wid, g])

    return sc_gather


_sc_gather = _sc_gather_factory()


def _tc_reduce_body(n_ref, u_ref, v_ref, w_ref, out_ref):
    i = pl.program_id(0)
    n3 = n_ref[0]
    u2 = u_ref[0, 0]
    v2 = v_ref[0, 0]
    wv = w_ref[0]
    pos = jnp.sum(u2 * v2, axis=1)
    coef = 1.5 - 0.5 * jnp.sign(wv)
    neg = jnp.sum(n3 * u2[None, :, :], axis=2)
    s = coef * pos - neg
    sp = jnp.maximum(-s, 0.0) + jnp.log1p(jnp.exp(-jnp.abs(s)))
    reg = jnp.sum(u2 * u2) + jnp.sum(v2 * v2) + jnp.sum(n3 * n3)
    partial = jnp.sum(sp) + REG * reg

    @pl.when(i == 0)
    def _():
        out_ref[0, 0] = 0.0

    out_ref[0, 0] += partial


def kernel(u, v, n, w, E, E2, z, edge_index, W0, b0, W1, b1,
           attn_W, attn_b, q_W):
    del E, E2, edge_index, W0, b0, W1, b1, attn_W, attn_b, q_W
    u = u.astype(jnp.int32)
    v = v.astype(jnp.int32)
    n = n.astype(jnp.int32)
    nt = n.reshape(NW, BPW, NEG).transpose(0, 2, 1).reshape(NW, NEG * BPW)
    idx = jnp.concatenate(
        [nt, u.reshape(NW, BPW), v.reshape(NW, BPW),
         jnp.zeros((NW, 2 * CHUNK), jnp.int32)], axis=1,
    ).reshape(NW, NCH, CHUNK)

    gathered = _sc_gather(idx, z)

    out = pl.pallas_call(
        _tc_reduce_body,
        grid=(NW,),
        in_specs=[
            pl.BlockSpec((1, NCH_N, CHUNK, DIM), lambda i: (i, 0, 0, 0)),
            pl.BlockSpec((1, 1, CHUNK, DIM), lambda i: (i, NCH_N, 0, 0)),
            pl.BlockSpec((1, 1, CHUNK, DIM), lambda i: (i, NCH_N + 1, 0, 0)),
            pl.BlockSpec((1, CHUNK), lambda i: (i, 0)),
        ],
        out_specs=pl.BlockSpec((1, 1), lambda i: (0, 0)),
        out_shape=jax.ShapeDtypeStruct((1, 1), jnp.float32),
    )(gathered, gathered, gathered, w.reshape(NW, BPW))
    return out[0, 0]


# speedup vs baseline: 1.7858x; 1.7858x over previous
import functools

import jax
import jax.numpy as jnp
from jax import lax
from jax.experimental import pallas as pl
from jax.experimental.pallas import tpu as pltpu
from jax.experimental.pallas import tpu_sc as plsc

M = 30000
NV = 20000
NN = M + NV
DIM = 64
B = 4096
NEG = 40
REG = 1e-4

NW = 32
BPW = B // NW
CHUNK = 128
NCH_N = NEG
NCH = NCH_N + 4
NB = 4


def _sc_gather_factory():
    mesh = plsc.VectorSubcoreMesh(core_axis_name="c", subcore_axis_name="s")

    @functools.partial(
        pl.kernel,
        out_type=jax.ShapeDtypeStruct((NW, NCH, CHUNK, DIM), jnp.float32),
        mesh=mesh,
        scratch_types=[
            pltpu.VMEM((NCH, CHUNK), jnp.int32),
            pltpu.VMEM((NB, CHUNK, DIM), jnp.float32),
            pltpu.SemaphoreType.DMA((NB,)),
        ],
        compiler_params=pltpu.CompilerParams(use_tc_tiling_on_sc=False),
    )
    def sc_gather(idx_hbm, z_hbm, out_hbm, idx_v, rows_v, gsem):
        wid = lax.axis_index("s") * 2 + lax.axis_index("c")
        pltpu.sync_copy(idx_hbm.at[wid], idx_v)

        def start(g, b):
            pltpu.async_copy(z_hbm.at[idx_v.at[g]], rows_v.at[b], gsem.at[b])

        def wait(b):
            pltpu.make_async_copy(
                z_hbm.at[idx_v.at[0]], rows_v.at[b], gsem.at[b]
            ).wait()

        for b in range(NB):
            start(b, b)

        def group(i, carry):
            for b in range(NB):
                g = i * NB + b
                wait(b)
                pltpu.sync_copy(rows_v.at[b], out_hbm.at[wid, g])
                start(g + NB, b)
            return carry

        lax.fori_loop(0, NCH_N // NB, group, 0, unroll=False)

        for b in range(NB):
            g = NCH_N + b
            wait(b)
            if g < NCH_N + 2:
                pltpu.sync_copy(rows_v.at[b], out_hbm.at[wid, g])

    return sc_gather


_sc_gather = _sc_gather_factory()


def _tc_reduce_body(n_ref, u_ref, v_ref, w_ref, out_ref):
    i = pl.program_id(0)
    n3 = n_ref[0]
    u2 = u_ref[0, 0]
    v2 = v_ref[0, 0]
    wv = w_ref[0, 0]
    pos = jnp.sum(u2 * v2, axis=1)
    coef = 1.5 - 0.5 * jnp.sign(wv)
    neg = jnp.sum(n3 * u2[None, :, :], axis=2)
    s = coef * pos - neg
    sp = jnp.maximum(-s, 0.0) + jnp.log1p(jnp.exp(-jnp.abs(s)))
    reg = jnp.sum(u2 * u2) + jnp.sum(v2 * v2) + jnp.sum(n3 * n3)
    partial = jnp.sum(sp) + REG * reg

    @pl.when(i == 0)
    def _():
        out_ref[...] = jnp.zeros_like(out_ref)

    out_ref[...] += partial.reshape(1, 1)


def kernel(u, v, n, w, E, E2, z, edge_index, W0, b0, W1, b1,
           attn_W, attn_b, q_W):
    del E, E2, edge_index, W0, b0, W1, b1, attn_W, attn_b, q_W
    u = u.astype(jnp.int32)
    v = v.astype(jnp.int32)
    n = n.astype(jnp.int32)
    nt = n.reshape(NW, BPW, NEG).transpose(0, 2, 1).reshape(NW, NEG * BPW)
    idx = jnp.concatenate(
        [nt, u.reshape(NW, BPW), v.reshape(NW, BPW),
         jnp.zeros((NW, 2 * CHUNK), jnp.int32)], axis=1,
    ).reshape(NW, NCH, CHUNK)

    gathered = _sc_gather(idx, z)

    out = pl.pallas_call(
        _tc_reduce_body,
        grid=(NW,),
        in_specs=[
            pl.BlockSpec((1, NCH_N, CHUNK, DIM), lambda i: (i, 0, 0, 0)),
            pl.BlockSpec((1, 1, CHUNK, DIM), lambda i: (i, NCH_N, 0, 0)),
            pl.BlockSpec((1, 1, CHUNK, DIM), lambda i: (i, NCH_N + 1, 0, 0)),
            pl.BlockSpec((1, 1, CHUNK), lambda i: (i, 0, 0)),
        ],
        out_specs=pl.BlockSpec((1, 1), lambda i: (0, 0)),
        out_shape=jax.ShapeDtypeStruct((1, 1), jnp.float32),
    )(gathered, gathered, gathered, w.reshape(NW, 1, BPW))
    return out[0, 0]


# speedup vs baseline: 3.2245x; 1.8057x over previous
import functools

import jax
import jax.numpy as jnp
from jax import lax
from jax.experimental import pallas as pl
from jax.experimental.pallas import tpu as pltpu
from jax.experimental.pallas import tpu_sc as plsc

M = 30000
NV = 20000
NN = M + NV
DIM = 64
B = 4096
NEG = 40
REG = 1e-4

NW = 32
BPW = B // NW
CHUNK = 128
NCH_N = NEG
NCH = NCH_N + 2
NB = 8


def _sc_gather_factory():
    mesh = plsc.VectorSubcoreMesh(core_axis_name="c", subcore_axis_name="s")

    @functools.partial(
        pl.kernel,
        out_type=jax.ShapeDtypeStruct((NW, NCH, CHUNK, DIM), jnp.float32),
        mesh=mesh,
        scratch_types=[
            pltpu.VMEM((NCH, CHUNK), jnp.int32),
            pltpu.VMEM((NB, CHUNK, DIM), jnp.float32),
            pltpu.SemaphoreType.DMA((NB,)),
            pltpu.SemaphoreType.DMA((NB,)),
        ],
        compiler_params=pltpu.CompilerParams(use_tc_tiling_on_sc=False),
    )
    def sc_gather(idx_hbm, z_hbm, out_hbm, idx_v, rows_v, gsem, wsem):
        wid = lax.axis_index("s") * 2 + lax.axis_index("c")
        pltpu.sync_copy(idx_hbm.at[wid], idx_v)

        def start_gather(g, b):
            pltpu.async_copy(z_hbm.at[idx_v.at[g]], rows_v.at[b], gsem.at[b])

        def wait_gather(b):
            pltpu.make_async_copy(
                z_hbm.at[idx_v.at[0]], rows_v.at[b], gsem.at[b]
            ).wait()

        def start_wb(g, b):
            pltpu.async_copy(rows_v.at[b], out_hbm.at[wid, g], wsem.at[b])

        def wait_wb(b):
            pltpu.make_async_copy(
                rows_v.at[b], out_hbm.at[wid, 0], wsem.at[b]
            ).wait()

        def process(c, b, guarded):
            b2 = (b + NB - 2) % NB
            if isinstance(c, int):
                if c >= 2:
                    wait_wb(b2)
                    if c + NB - 2 < NCH:
                        start_gather(c + NB - 2, b2)
            else:
                wait_wb(b2)
                if guarded:
                    @pl.when(c + NB - 2 < NCH)
                    def _():
                        start_gather(c + NB - 2, b2)
                else:
                    start_gather(c + NB - 2, b2)
            wait_gather(b)
            start_wb(c, b)

        for b in range(NB):
            start_gather(b, b)
        for c in range(NB):
            process(c, c, False)

        def group(i, carry):
            for b in range(NB):
                c = i * NB + b
                process(c, b, True)
            return carry

        ngroups = (NCH - NB) // NB
        lax.fori_loop(1, 1 + ngroups, group, 0, unroll=False)
        for c in range(NB * (1 + ngroups), NCH):
            process(c, c % NB, False)
        wait_wb((NCH - 2) % NB)
        wait_wb((NCH - 1) % NB)

    return sc_gather


_sc_gather = _sc_gather_factory()


def _tc_reduce_body(n_ref, u_ref, v_ref, w_ref, out_ref):
    i = pl.program_id(0)
    n3 = n_ref[0]
    u2 = u_ref[0, 0]
    v2 = v_ref[0, 0]
    wv = w_ref[0, 0]
    pos = jnp.sum(u2 * v2, axis=1)
    coef = 1.5 - 0.5 * jnp.sign(wv)
    neg = jnp.sum(n3 * u2[None, :, :], axis=2)
    s = coef * pos - neg
    sp = jnp.maximum(-s, 0.0) + jnp.log1p(jnp.exp(-jnp.abs(s)))
    reg = jnp.sum(u2 * u2) + jnp.sum(v2 * v2) + jnp.sum(n3 * n3)
    partial = jnp.sum(sp) + REG * reg

    @pl.when(i == 0)
    def _():
        out_ref[...] = jnp.zeros_like(out_ref)

    out_ref[...] += partial.reshape(1, 1)


def kernel(u, v, n, w, E, E2, z, edge_index, W0, b0, W1, b1,
           attn_W, attn_b, q_W):
    del E, E2, edge_index, W0, b0, W1, b1, attn_W, attn_b, q_W
    u = u.astype(jnp.int32)
    v = v.astype(jnp.int32)
    n = n.astype(jnp.int32)
    nt = n.reshape(NW, BPW, NEG).transpose(0, 2, 1).reshape(NW, NEG * BPW)
    idx = jnp.concatenate(
        [nt, u.reshape(NW, BPW), v.reshape(NW, BPW)], axis=1,
    ).reshape(NW, NCH, CHUNK)

    gathered = _sc_gather(idx, z)

    out = pl.pallas_call(
        _tc_reduce_body,
        grid=(NW,),
        in_specs=[
            pl.BlockSpec((1, NCH_N, CHUNK, DIM), lambda i: (i, 0, 0, 0)),
            pl.BlockSpec((1, 1, CHUNK, DIM), lambda i: (i, NCH_N, 0, 0)),
            pl.BlockSpec((1, 1, CHUNK, DIM), lambda i: (i, NCH_N + 1, 0, 0)),
            pl.BlockSpec((1, 1, CHUNK), lambda i: (i, 0, 0)),
        ],
        out_specs=pl.BlockSpec((1, 1), lambda i: (0, 0)),
        out_shape=jax.ShapeDtypeStruct((1, 1), jnp.float32),
    )(gathered, gathered, gathered, w.reshape(NW, 1, BPW))
    return out[0, 0]


# speedup vs baseline: 3.7571x; 1.1652x over previous
import functools

import jax
import jax.numpy as jnp
from jax import lax
from jax.experimental import pallas as pl
from jax.experimental.pallas import tpu as pltpu
from jax.experimental.pallas import tpu_sc as plsc

M = 30000
NV = 20000
NN = M + NV
DIM = 64
B = 4096
NEG = 40
REG = 1e-4

NW = 32
BPW = B // NW
CHUNK = 128
NCH = NEG + 2
SGB = 16
NSG = BPW // SGB
SGC = SGB * NEG // CHUNK
L = 16
NC = DIM // L


def _sc_factory():
    mesh = plsc.VectorSubcoreMesh(core_axis_name="c", subcore_axis_name="s")

    @functools.partial(
        pl.kernel,
        out_type=(
            jax.ShapeDtypeStruct((NW, BPW * NEG // 8, 128), jnp.float32),
            jax.ShapeDtypeStruct((NW, 17, 128), jnp.float32),
        ),
        mesh=mesh,
        scratch_types=[
            pltpu.VMEM((NCH, CHUNK), jnp.int32),
            pltpu.VMEM((2, SGB * NEG, DIM), jnp.float32),
            pltpu.VMEM((CHUNK, DIM), jnp.float32),
            pltpu.VMEM((CHUNK, DIM), jnp.float32),
            pltpu.VMEM((2, SGB * NEG // 8, 128), jnp.float32),
            pltpu.VMEM((17, 128), jnp.float32),
            pltpu.SemaphoreType.DMA((2,)),
            pltpu.SemaphoreType.DMA((2,)),
            pltpu.SemaphoreType.DMA,
        ],
        compiler_params=pltpu.CompilerParams(use_tc_tiling_on_sc=False),
    )
    def sc_bpr(idx_hbm, z_hbm, np_hbm, uv_hbm, idx_v, rows_v, u_v, v_v,
               psum_v, uvp_v, gsem, wsem, usem):
        wid = lax.axis_index("s") * 2 + lax.axis_index("c")
        pltpu.sync_copy(idx_hbm.at[wid], idx_v)

        def start_sg_gathers(sg, slot):
            for k in range(SGC):
                pltpu.async_copy(
                    z_hbm.at[idx_v.at[sg * SGC + k]],
                    rows_v.at[slot, pl.ds(k * CHUNK, CHUNK)],
                    gsem.at[slot])

        def wait_sg_gathers(slot):
            for _ in range(SGC):
                pltpu.make_async_copy(
                    z_hbm.at[idx_v.at[0]],
                    rows_v.at[slot, pl.ds(0, CHUNK)],
                    gsem.at[slot]).wait()

        pltpu.async_copy(z_hbm.at[idx_v.at[NEG]], u_v, usem)
        pltpu.async_copy(z_hbm.at[idx_v.at[NEG + 1]], v_v, usem)
        start_sg_gathers(0, 0)
        start_sg_gathers(1, 1)
        pltpu.make_async_copy(z_hbm.at[idx_v.at[0]], u_v, usem).wait()
        pltpu.make_async_copy(z_hbm.at[idx_v.at[0]], v_v, usem).wait()

        zero = jnp.zeros((L,), jnp.float32)
        for k in range(8):
            uvp_v[16, pl.ds(k * L, L)] = zero

        def uv_body(b, racc):
            r0, r1, r2, r3 = racc
            uc = [u_v[b, pl.ds(c * L, L)] for c in range(NC)]
            vc = [v_v[b, pl.ds(c * L, L)] for c in range(NC)]
            p = uc[0] * vc[0] + uc[1] * vc[1] + uc[2] * vc[2] + uc[3] * vc[3]
            r0 = r0 + uc[0] * uc[0] + vc[0] * vc[0]
            r1 = r1 + uc[1] * uc[1] + vc[1] * vc[1]
            r2 = r2 + uc[2] * uc[2] + vc[2] * vc[2]
            r3 = r3 + uc[3] * uc[3] + vc[3] * vc[3]
            uvp_v[b // 8, pl.ds((b % 8) * L, L)] = p
            return (r0, r1, r2, r3)

        racc = lax.fori_loop(0, BPW, uv_body, (zero, zero, zero, zero),
                             unroll=False)

        for sg in range(NSG):
            slot = sg % 2
            if sg >= 2:
                pltpu.make_async_copy(
                    psum_v.at[slot], np_hbm.at[wid, pl.ds(0, SGB * NEG // 8)],
                    wsem.at[slot]).wait()
            wait_sg_gathers(slot)

            def b_body(bl, racc, _slot=slot, _sg=sg):
                b = _sg * SGB + bl
                uc = [u_v[b, pl.ds(c * L, L)] for c in range(NC)]

                def j_body(j, racc2):
                    r0, r1, r2, r3 = racc2
                    row = bl * NEG + j
                    nc = [rows_v[_slot, row, pl.ds(c * L, L)]
                          for c in range(NC)]
                    p = (nc[0] * uc[0] + nc[1] * uc[1]
                         + nc[2] * uc[2] + nc[3] * uc[3])
                    r0 = r0 + nc[0] * nc[0]
                    r1 = r1 + nc[1] * nc[1]
                    r2 = r2 + nc[2] * nc[2]
                    r3 = r3 + nc[3] * nc[3]
                    psum_v[_slot, row // 8, pl.ds((row % 8) * L, L)] = p
                    return (r0, r1, r2, r3)

                return lax.fori_loop(0, NEG, j_body, racc, unroll=2)

            racc = lax.fori_loop(0, SGB, b_body, racc, unroll=False)
            pltpu.async_copy(
                psum_v.at[slot],
                np_hbm.at[wid, pl.ds(sg * (SGB * NEG // 8), SGB * NEG // 8)],
                wsem.at[slot])
            if sg + 2 < NSG:
                start_sg_gathers(sg + 2, slot)

        r = racc[0] + racc[1] + racc[2] + racc[3]
        uvp_v[16, pl.ds(0, L)] = r
        for slot in range(2):
            pltpu.make_async_copy(
                psum_v.at[slot], np_hbm.at[wid, pl.ds(0, SGB * NEG // 8)],
                wsem.at[slot]).wait()
        pltpu.sync_copy(uvp_v, uv_hbm.at[wid])

    return sc_bpr


_sc_bpr = _sc_factory()


def _tc_reduce_body(np_ref, uv_ref, w_ref, out_ref):
    i = pl.program_id(0)
    X = np_ref[0]
    neg = jnp.sum(X.reshape(BPW * NEG // 8, 8, L), axis=2)
    neg = neg.reshape(BPW, NEG)
    U = uv_ref[0]
    pos = jnp.sum(U[:16].reshape(BPW // 8, 8, L), axis=2).reshape(BPW)
    reg = jnp.sum(U[16, :L])
    wv = w_ref[0, 0]
    coef = 1.5 - 0.5 * jnp.sign(wv)
    s = (coef * pos)[:, None] - neg
    sp = jnp.maximum(-s, 0.0) + jnp.log1p(jnp.exp(-jnp.abs(s)))
    partial = jnp.sum(sp) + REG * reg

    @pl.when(i == 0)
    def _():
        out_ref[...] = jnp.zeros_like(out_ref)

    out_ref[...] += partial.reshape(1, 1)


def kernel(u, v, n, w, E, E2, z, edge_index, W0, b0, W1, b1,
           attn_W, attn_b, q_W):
    del E, E2, edge_index, W0, b0, W1, b1, attn_W, attn_b, q_W
    u = u.astype(jnp.int32)
    v = v.astype(jnp.int32)
    n = n.astype(jnp.int32)
    idx = jnp.concatenate(
        [n.reshape(NW, BPW * NEG), u.reshape(NW, BPW), v.reshape(NW, BPW)],
        axis=1,
    ).reshape(NW, NCH, CHUNK)

    np_out, uv_out = _sc_bpr(idx, z)

    out = pl.pallas_call(
        _tc_reduce_body,
        grid=(NW,),
        in_specs=[
            pl.BlockSpec((1, BPW * NEG // 8, 128), lambda i: (i, 0, 0)),
            pl.BlockSpec((1, 17, 128), lambda i: (i, 0, 0)),
            pl.BlockSpec((1, 1, BPW), lambda i: (i, 0, 0)),
        ],
        out_specs=pl.BlockSpec((1, 1), lambda i: (0, 0)),
        out_shape=jax.ShapeDtypeStruct((1, 1), jnp.float32),
    )(np_out, uv_out, w.reshape(NW, 1, BPW))
    return out[0, 0]


# speedup vs baseline: 4.4722x; 1.1903x over previous
import functools

import jax
import jax.numpy as jnp
from jax import lax
from jax.experimental import pallas as pl
from jax.experimental.pallas import tpu as pltpu
from jax.experimental.pallas import tpu_sc as plsc

M = 30000
NV = 20000
NN = M + NV
DIM = 64
B = 4096
NEG = 40
REG = 1e-4

NW = 32
BPW = B // NW
CHUNK = 128
NCH = NEG + 2
SGB = 16
NSG = BPW // SGB
SGC = SGB * NEG // CHUNK
L = 16
NC = DIM // L


def _sc_factory():
    mesh = plsc.VectorSubcoreMesh(core_axis_name="c", subcore_axis_name="s")

    @functools.partial(
        pl.kernel,
        out_type=(
            jax.ShapeDtypeStruct((NW, BPW * NEG // 8, 128), jnp.float32),
            jax.ShapeDtypeStruct((NW, 17, 128), jnp.float32),
        ),
        mesh=mesh,
        scratch_types=[
            pltpu.VMEM((NCH, CHUNK), jnp.int32),
            pltpu.VMEM((2, SGB * NEG, DIM), jnp.float32),
            pltpu.VMEM((CHUNK, DIM), jnp.float32),
            pltpu.VMEM((CHUNK, DIM), jnp.float32),
            pltpu.VMEM((2, SGB * NEG // 8, 128), jnp.float32),
            pltpu.VMEM((17, 128), jnp.float32),
            pltpu.SemaphoreType.DMA((2,)),
            pltpu.SemaphoreType.DMA((2,)),
            pltpu.SemaphoreType.DMA,
        ],
        compiler_params=pltpu.CompilerParams(use_tc_tiling_on_sc=False),
    )
    def sc_bpr(nidx_hbm, uidx_hbm, vidx_hbm, z_hbm, np_hbm, uv_hbm,
               idx_v, rows_v, u_v, v_v, psum_v, uvp_v, gsem, wsem, usem):
        wid = lax.axis_index("s") * 2 + lax.axis_index("c")
        pltpu.sync_copy(nidx_hbm.at[wid], idx_v.at[pl.ds(0, NEG)])
        pltpu.sync_copy(uidx_hbm.at[wid], idx_v.at[NEG])
        pltpu.sync_copy(vidx_hbm.at[wid], idx_v.at[NEG + 1])

        def start_sg_gathers(sg, slot):
            for k in range(SGC):
                pltpu.async_copy(
                    z_hbm.at[idx_v.at[sg * SGC + k]],
                    rows_v.at[slot, pl.ds(k * CHUNK, CHUNK)],
                    gsem.at[slot])

        def wait_sg_gathers(slot):
            for _ in range(SGC):
                pltpu.make_async_copy(
                    z_hbm.at[idx_v.at[0]],
                    rows_v.at[slot, pl.ds(0, CHUNK)],
                    gsem.at[slot]).wait()

        pltpu.async_copy(z_hbm.at[idx_v.at[NEG]], u_v, usem)
        pltpu.async_copy(z_hbm.at[idx_v.at[NEG + 1]], v_v, usem)
        start_sg_gathers(0, 0)
        start_sg_gathers(1, 1)
        pltpu.make_async_copy(z_hbm.at[idx_v.at[0]], u_v, usem).wait()
        pltpu.make_async_copy(z_hbm.at[idx_v.at[0]], v_v, usem).wait()

        zero = jnp.zeros((L,), jnp.float32)
        for k in range(8):
            uvp_v[16, pl.ds(k * L, L)] = zero

        def uv_body(b, racc):
            r0, r1, r2, r3 = racc
            uc = [u_v[b, pl.ds(c * L, L)] for c in range(NC)]
            vc = [v_v[b, pl.ds(c * L, L)] for c in range(NC)]
            p = uc[0] * vc[0] + uc[1] * vc[1] + uc[2] * vc[2] + uc[3] * vc[3]
            r0 = r0 + uc[0] * uc[0] + vc[0] * vc[0]
            r1 = r1 + uc[1] * uc[1] + vc[1] * vc[1]
            r2 = r2 + uc[2] * uc[2] + vc[2] * vc[2]
            r3 = r3 + uc[3] * uc[3] + vc[3] * vc[3]
            uvp_v[b // 8, pl.ds((b % 8) * L, L)] = p
            return (r0, r1, r2, r3)

        racc = lax.fori_loop(0, BPW, uv_body, (zero, zero, zero, zero),
                             unroll=False)

        for sg in range(NSG):
            slot = sg % 2
            if sg >= 2:
                pltpu.make_async_copy(
                    psum_v.at[slot], np_hbm.at[wid, pl.ds(0, SGB * NEG // 8)],
                    wsem.at[slot]).wait()
            wait_sg_gathers(slot)

            def b_body(bl, racc, _slot=slot, _sg=sg):
                b = _sg * SGB + bl
                uc = [u_v[b, pl.ds(c * L, L)] for c in range(NC)]

                def j_body(j, racc2):
                    r0, r1, r2, r3 = racc2
                    row = bl * NEG + j
                    nc = [rows_v[_slot, row, pl.ds(c * L, L)]
                          for c in range(NC)]
                    p = (nc[0] * uc[0] + nc[1] * uc[1]
                         + nc[2] * uc[2] + nc[3] * uc[3])
                    r0 = r0 + nc[0] * nc[0]
                    r1 = r1 + nc[1] * nc[1]
                    r2 = r2 + nc[2] * nc[2]
                    r3 = r3 + nc[3] * nc[3]
                    psum_v[_slot, row // 8, pl.ds((row % 8) * L, L)] = p
                    return (r0, r1, r2, r3)

                return lax.fori_loop(0, NEG, j_body, racc, unroll=2)

            racc = lax.fori_loop(0, SGB, b_body, racc, unroll=False)
            pltpu.async_copy(
                psum_v.at[slot],
                np_hbm.at[wid, pl.ds(sg * (SGB * NEG // 8), SGB * NEG // 8)],
                wsem.at[slot])
            if sg + 2 < NSG:
                start_sg_gathers(sg + 2, slot)

        r = racc[0] + racc[1] + racc[2] + racc[3]
        uvp_v[16, pl.ds(0, L)] = r
        for slot in range(2):
            pltpu.make_async_copy(
                psum_v.at[slot], np_hbm.at[wid, pl.ds(0, SGB * NEG // 8)],
                wsem.at[slot]).wait()
        pltpu.sync_copy(uvp_v, uv_hbm.at[wid])

    return sc_bpr


_sc_bpr = _sc_factory()


def _mm(a, b):
    return jax.lax.dot_general(a, b, (((1,), (0,)), ((), ())),
                               preferred_element_type=jnp.float32)


def _tc_reduce_body(np_ref, uv_ref, w_ref, out_ref):
    i = pl.program_id(0)
    f32 = jnp.float32

    def iota(shape, d):
        return lax.broadcasted_iota(jnp.int32, shape, d)

    S = (iota((128, 8), 0) // L == iota((128, 8), 1)).astype(f32)
    X = np_ref[0]
    neg = _mm(X, S)
    U = uv_ref[0]
    pos = _mm(U[:16], S)
    reg = jnp.sum(U[16, :L])
    wv = w_ref[0]
    cpq = (1.5 - 0.5 * jnp.sign(wv)) * pos
    E1 = (iota((128, 16), 0) // 8 == iota((128, 16), 1)).astype(f32)
    Msel = (iota((128, 8), 0) % 8 == iota((128, 8), 1)).astype(f32)
    cpcol = _mm(_mm(E1, cpq) * Msel, jnp.ones((8, 1), f32))
    M3 = (iota((640, 128), 0) // (NEG // 8) == iota((640, 128), 1)).astype(f32)
    cp = _mm(M3, cpcol)
    s = cp - neg
    sp = jnp.maximum(-s, 0.0) + jnp.log1p(jnp.exp(-jnp.abs(s)))
    partial = jnp.sum(sp) + REG * reg

    @pl.when(i == 0)
    def _():
        out_ref[...] = jnp.zeros_like(out_ref)

    out_ref[...] += partial.reshape(1, 1)


def kernel(u, v, n, w, E, E2, z, edge_index, W0, b0, W1, b1,
           attn_W, attn_b, q_W):
    del E, E2, edge_index, W0, b0, W1, b1, attn_W, attn_b, q_W
    u = u.astype(jnp.int32)
    v = v.astype(jnp.int32)
    n = n.astype(jnp.int32)
    np_out, uv_out = _sc_bpr(
        n.reshape(NW, NEG, CHUNK), u.reshape(NW, CHUNK),
        v.reshape(NW, CHUNK), z)

    out = pl.pallas_call(
        _tc_reduce_body,
        grid=(NW,),
        in_specs=[
            pl.BlockSpec((1, BPW * NEG // 8, 128), lambda i: (i, 0, 0)),
            pl.BlockSpec((1, 17, 128), lambda i: (i, 0, 0)),
            pl.BlockSpec((1, 16, 8), lambda i: (i, 0, 0)),
        ],
        out_specs=pl.BlockSpec((1, 1), lambda i: (0, 0)),
        out_shape=jax.ShapeDtypeStruct((1, 1), jnp.float32),
    )(np_out, uv_out, w.reshape(NW, 16, 8))
    return out[0, 0]
